# baseline (speedup 1.0000x reference)
import functools

import jax
import jax.numpy as jnp
from jax import lax
from jax.experimental import pallas as pl
from jax.experimental.pallas import tpu as pltpu
from jax.experimental.pallas import tpu_sc as plsc

B_, N_, C_ = 4, 8192, 64
S_, K_ = 1024, 24
D2_ = 2 * C_
NSC, NSUB = 2, 16
NW = NSC * NSUB
NROWS = B_ * S_ * K_
NALL = NROWS + B_ * S_
CNT = float(B_ * S_ * K_)
NSTD = S_ * K_ * C_


def _fps_body(xyz_ref, nx_ref, ny_ref, nz_ref, gidx_ref, dist_ref):
    io_s = lax.broadcasted_iota(jnp.int32, (64, 128), 0)
    io_l = lax.broadcasted_iota(jnp.int32, (64, 128), 1)
    pos = io_s * 128 + io_l
    io_p = lax.broadcasted_iota(jnp.int32, (1, 1024), 1)

    dist_ref[...] = jnp.full((4, 64, 128), 1e10, dtype=jnp.float32)

    def step(s, far):
        sel = io_p == s
        io128 = lax.broadcasted_iota(jnp.int32, (1, 128), 1)
        newfar = []
        for b in range(4):
            f = far[b]
            fs = f // 128
            fl = f % 128
            lm = io128 == fl
            cx = jnp.sum(jnp.where(lm, xyz_ref[b, 0, pl.ds(fs, 1), :], 0.0))
            cy = jnp.sum(jnp.where(lm, xyz_ref[b, 1, pl.ds(fs, 1), :], 0.0))
            cz = jnp.sum(jnp.where(lm, xyz_ref[b, 2, pl.ds(fs, 1), :], 0.0))
            nx_ref[b:b + 1, :] = jnp.where(sel, cx, nx_ref[b:b + 1, :])
            ny_ref[b:b + 1, :] = jnp.where(sel, cy, ny_ref[b:b + 1, :])
            nz_ref[b:b + 1, :] = jnp.where(sel, cz, nz_ref[b:b + 1, :])
            gidx_ref[b:b + 1, :] = jnp.where(sel, f + b * N_,
                                             gidx_ref[b:b + 1, :])
            dx = xyz_ref[b, 0] - cx
            dy = xyz_ref[b, 1] - cy
            dz = xyz_ref[b, 2] - cz
            d = (dx * dx + dy * dy) + dz * dz
            dist = jnp.minimum(dist_ref[b], d)
            dist_ref[b] = dist
            m = jnp.max(dist)
            cand = jnp.where(dist == m, pos, jnp.int32(2 ** 30))
            newfar.append(jnp.min(cand))
        return tuple(newfar)

    lax.fori_loop(0, S_, step, (jnp.int32(0),) * 4)


def _run_fps(xyz_v):
    f32 = jnp.float32
    return pl.pallas_call(
        _fps_body,
        out_shape=(
            jax.ShapeDtypeStruct((4, 1024), f32),
            jax.ShapeDtypeStruct((4, 1024), f32),
            jax.ShapeDtypeStruct((4, 1024), f32),
            jax.ShapeDtypeStruct((4, 1024), jnp.int32),
        ),
        scratch_shapes=[pltpu.VMEM((4, 64, 128), f32)],
    )(xyz_v)


SQ = 256


def _knn_body(q_ref, xyz_ref, idx_ref):
    b = pl.program_id(0)
    q = q_ref[0]
    qx = q[:, 0:1]
    qy = q[:, 1:2]
    qz = q[:, 2:3]
    px = xyz_ref[0, 0:1, :]
    py = xyz_ref[0, 1:2, :]
    pz = xyz_ref[0, 2:3, :]
    qn = (qx * qx + qy * qy) + qz * qz
    pn = (px * px + py * py) + pz * pz
    bf = jnp.bfloat16
    f32 = jnp.float32
    qxb = qx.astype(bf).astype(f32)
    qyb = qy.astype(bf).astype(f32)
    qzb = qz.astype(bf).astype(f32)
    pxb = px.astype(bf).astype(f32)
    pyb = py.astype(bf).astype(f32)
    pzb = pz.astype(bf).astype(f32)
    qp = (qxb * pxb + qyb * pyb) + qzb * pzb
    d = (qn + pn) - 2.0 * qp
    li = lax.broadcasted_iota(jnp.int32, (SQ, N_), 1)
    boff = b * N_
    for j in range(K_):
        m = jnp.min(d, axis=1, keepdims=True)
        cand = jnp.where(d == m, li, jnp.int32(2 ** 30))
        a = jnp.min(cand, axis=1, keepdims=True)
        idx_ref[0, :, j:j + 1] = a + boff
        d = jnp.where(li == a, jnp.float32(3e38), d)


def _run_knn(new_xyz, xyz_t):
    nst = S_ // SQ
    return pl.pallas_call(
        _knn_body,
        grid=(B_, nst),
        in_specs=[
            pl.BlockSpec((1, SQ, 3), lambda b, st: (b, st, 0)),
            pl.BlockSpec((1, 3, N_), lambda b, st: (b, 0, 0)),
        ],
        out_specs=pl.BlockSpec((1, SQ, K_), lambda b, st: (b, st, 0)),
        out_shape=jax.ShapeDtypeStruct((B_, S_, K_), jnp.int32),
    )(new_xyz, xyz_t)


RPW = NALL // NW
CH = 800


def _sc_gather_body(table_hbm, idx_hbm, out_hbm, idx_v, rows_v, sem):
    wid = lax.axis_index("s") * NSC + lax.axis_index("c")
    base = wid * RPW
    for c in range(RPW // CH):
        off = base + c * CH
        pltpu.sync_copy(idx_hbm.at[pl.ds(off, CH)], idx_v)
        pltpu.async_copy(table_hbm.at[idx_v], rows_v, sem).wait()
        pltpu.sync_copy(rows_v, out_hbm.at[pl.ds(off, CH)])


def _run_sc_gather(table, idx_all):
    mesh = plsc.VectorSubcoreMesh(
        core_axis_name="c", subcore_axis_name="s",
        num_cores=NSC, num_subcores=NSUB)
    k = functools.partial(
        pl.kernel,
        out_type=jax.ShapeDtypeStruct((NALL, D2_), jnp.float32),
        mesh=mesh,
        scratch_types=[
            pltpu.VMEM((CH,), jnp.int32),
            pltpu.VMEM((CH, D2_), jnp.float32),
            pltpu.SemaphoreType.DMA,
        ],
    )(_sc_gather_body)
    return k(table, idx_all)


def _p2_body(g_ref, o_ref):
    st = pl.program_id(1)

    @pl.when(st == 0)
    def _():
        o_ref[...] = jnp.zeros_like(o_ref)

    g = g_ref[0]
    mean = jnp.mean(g, axis=1, keepdims=True)
    c = g - mean
    o_ref[...] = o_ref[...] + jnp.sum(c * c)


def _run_p2(grouped4):
    return pl.pallas_call(
        _p2_body,
        grid=(B_, 8),
        in_specs=[pl.BlockSpec((1, 128, K_, C_), lambda b, st: (b, st, 0, 0))],
        out_specs=pl.BlockSpec((1, 8, 128), lambda b, st: (b, 0, 0)),
        out_shape=jax.ShapeDtypeStruct((B_, 8, 128), jnp.float32),
    )(grouped4)


SCH = 128


def _p3_body(g_ref, np_ref, sdiv_ref, al_ref, be_ref, w1a_ref, w1b_ref,
             b1_ref, h1_ref, s1_ref, s2_ref):
    b = pl.program_id(0)
    sc = pl.program_id(1)

    @pl.when((b == 0) & (sc == 0))
    def _():
        s1_ref[...] = jnp.zeros_like(s1_ref)
        s2_ref[...] = jnp.zeros_like(s2_ref)

    g = g_ref[0]
    mean = jnp.mean(g, axis=1, keepdims=True)
    c = g - mean
    s = sdiv_ref[b]
    al = al_ref[...].reshape(1, 1, C_)
    be = be_ref[...].reshape(1, 1, C_)
    xg = al * (c / s) + be
    x1 = xg.reshape(SCH * K_, C_)
    x2 = jnp.broadcast_to(np_ref[0][:, None, :], (SCH, K_, C_)).reshape(
        SCH * K_, C_)
    h1 = (jnp.dot(x1, w1a_ref[...], preferred_element_type=jnp.float32)
          + jnp.dot(x2, w1b_ref[...], preferred_element_type=jnp.float32)
          ) + b1_ref[...]
    h1_ref[...] = h1
    s1_ref[...] = s1_ref[...] + jnp.sum(h1, axis=0, keepdims=True)
    s2_ref[...] = s2_ref[...] + jnp.sum(h1 * h1, axis=0, keepdims=True)


def _run_p3(grouped4, npnts, sdiv_t, al_t, be_t, w1a, w1b, b1_t):
    f32 = jnp.float32
    return pl.pallas_call(
        _p3_body,
        grid=(B_, S_ // SCH),
        in_specs=[
            pl.BlockSpec((1, SCH, K_, C_), lambda b, sc: (b, sc, 0, 0)),
            pl.BlockSpec((1, SCH, C_), lambda b, sc: (b, sc, 0)),
            pl.BlockSpec(memory_space=pltpu.SMEM),
            pl.BlockSpec((1, C_), lambda b, sc: (0, 0)),
            pl.BlockSpec((1, C_), lambda b, sc: (0, 0)),
            pl.BlockSpec((C_, D2_), lambda b, sc: (0, 0)),
            pl.BlockSpec((C_, D2_), lambda b, sc: (0, 0)),
            pl.BlockSpec((1, D2_), lambda b, sc: (0, 0)),
        ],
        out_specs=(
            pl.BlockSpec((SCH * K_, D2_),
                         lambda b, sc: (b * (S_ // SCH) + sc, 0)),
            pl.BlockSpec((1, D2_), lambda b, sc: (0, 0)),
            pl.BlockSpec((1, D2_), lambda b, sc: (0, 0)),
        ),
        out_shape=(
            jax.ShapeDtypeStruct((NROWS, D2_), f32),
            jax.ShapeDtypeStruct((1, D2_), f32),
            jax.ShapeDtypeStruct((1, D2_), f32),
        ),
    )(grouped4, npnts, sdiv_t, al_t, be_t, w1a, w1b, b1_t)


def _p4_body(h1_ref, a1_ref, c1_ref, w2a_ref, w2b_ref, b2a_ref, b2b_ref,
             h2a_ref, h2b_ref, t1a_ref, t1b_ref, t2a_ref, t2b_ref):
    i = pl.program_id(0)

    @pl.when(i == 0)
    def _():
        t1a_ref[...] = jnp.zeros_like(t1a_ref)
        t1b_ref[...] = jnp.zeros_like(t1b_ref)
        t2a_ref[...] = jnp.zeros_like(t2a_ref)
        t2b_ref[...] = jnp.zeros_like(t2b_ref)

    h1 = h1_ref[...]
    hn = a1_ref[...] * h1 + c1_ref[...]
    l = jnp.where(hn >= 0, hn, 0.01 * hn)
    h2a = jnp.dot(l, w2a_ref[...], preferred_element_type=jnp.float32) \
        + b2a_ref[...]
    h2b = jnp.dot(l, w2b_ref[...], preferred_element_type=jnp.float32) \
        + b2b_ref[...]
    h2a_ref[...] = h2a
    h2b_ref[...] = h2b
    t1a_ref[...] = t1a_ref[...] + jnp.sum(h2a, axis=0, keepdims=True)
    t1b_ref[...] = t1b_ref[...] + jnp.sum(h2b, axis=0, keepdims=True)
    t2a_ref[...] = t2a_ref[...] + jnp.sum(h2a * h2a, axis=0, keepdims=True)
    t2b_ref[...] = t2b_ref[...] + jnp.sum(h2b * h2b, axis=0, keepdims=True)


def _run_p4(h1, a1_t, c1_t, w2a, w2b, b2a_t, b2b_t):
    f32 = jnp.float32
    nt = NROWS // (SCH * K_)
    return pl.pallas_call(
        _p4_body,
        grid=(nt,),
        in_specs=[
            pl.BlockSpec((SCH * K_, D2_), lambda i: (i, 0)),
            pl.BlockSpec((1, D2_), lambda i: (0, 0)),
            pl.BlockSpec((1, D2_), lambda i: (0, 0)),
            pl.BlockSpec((D2_, C_), lambda i: (0, 0)),
            pl.BlockSpec((D2_, C_), lambda i: (0, 0)),
            pl.BlockSpec((1, C_), lambda i: (0, 0)),
            pl.BlockSpec((1, C_), lambda i: (0, 0)),
        ],
        out_specs=(
            pl.BlockSpec((SCH * K_, C_), lambda i: (i, 0)),
            pl.BlockSpec((SCH * K_, C_), lambda i: (i, 0)),
            pl.BlockSpec((1, C_), lambda i: (0, 0)),
            pl.BlockSpec((1, C_), lambda i: (0, 0)),
            pl.BlockSpec((1, C_), lambda i: (0, 0)),
            pl.BlockSpec((1, C_), lambda i: (0, 0)),
        ),
        out_shape=(
            jax.ShapeDtypeStruct((NROWS, C_), f32),
            jax.ShapeDtypeStruct((NROWS, C_), f32),
            jax.ShapeDtypeStruct((1, C_), f32),
            jax.ShapeDtypeStruct((1, C_), f32),
            jax.ShapeDtypeStruct((1, C_), f32),
            jax.ShapeDtypeStruct((1, C_), f32),
        ),
    )(h1, a1_t, c1_t, w2a, w2b, b2a_t, b2b_t)


def _p5_body(h2a_ref, h2b_ref, a2a_ref, c2a_ref, a2b_ref, c2b_ref,
             g_ref, np_ref, sdiv_ref, al_ref, be_ref, o1_ref, o2_ref):
    b = pl.program_id(0)
    g = g_ref[0]
    mean = jnp.mean(g, axis=1, keepdims=True)
    c = g - mean
    s = sdiv_ref[b]
    al = al_ref[...].reshape(1, 1, C_)
    be = be_ref[...].reshape(1, 1, C_)
    xg = al * (c / s) + be
    x1 = xg.reshape(SCH * K_, C_)
    x2 = jnp.broadcast_to(np_ref[0][:, None, :], (SCH, K_, C_)).reshape(
        SCH * K_, C_)
    ra = a2a_ref[...] * h2a_ref[...] + c2a_ref[...] + x1
    rb = a2b_ref[...] * h2b_ref[...] + c2b_ref[...] + x2
    la = jnp.where(ra >= 0, ra, 0.01 * ra).reshape(SCH, K_, C_)
    lb = jnp.where(rb >= 0, rb, 0.01 * rb).reshape(SCH, K_, C_)
    o1_ref[0] = jnp.max(la, axis=1)
    o2_ref[0] = jnp.max(lb, axis=1)


def _run_p5(h2a, h2b, a2a_t, c2a_t, a2b_t, c2b_t, grouped4, npnts, sdiv_t,
            al_t, be_t):
    f32 = jnp.float32
    nsc = S_ // SCH
    small = lambda b, sc: (0, 0)
    return pl.pallas_call(
        _p5_body,
        grid=(B_, nsc),
        in_specs=[
            pl.BlockSpec((SCH * K_, C_), lambda b, sc: (b * nsc + sc, 0)),
            pl.BlockSpec((SCH * K_, C_), lambda b, sc: (b * nsc + sc, 0)),
            pl.BlockSpec((1, C_), small),
            pl.BlockSpec((1, C_), small),
            pl.BlockSpec((1, C_), small),
            pl.BlockSpec((1, C_), small),
            pl.BlockSpec((1, SCH, K_, C_), lambda b, sc: (b, sc, 0, 0)),
            pl.BlockSpec((1, SCH, C_), lambda b, sc: (b, sc, 0)),
            pl.BlockSpec(memory_space=pltpu.SMEM),
            pl.BlockSpec((1, C_), small),
            pl.BlockSpec((1, C_), small),
        ],
        out_specs=(
            pl.BlockSpec((1, SCH, C_), lambda b, sc: (b, sc, 0)),
            pl.BlockSpec((1, SCH, C_), lambda b, sc: (b, sc, 0)),
        ),
        out_shape=(
            jax.ShapeDtypeStruct((B_, S_, C_), f32),
            jax.ShapeDtypeStruct((B_, S_, C_), f32),
        ),
    )(h2a, h2b, a2a_t, c2a_t, a2b_t, c2b_t, grouped4, npnts, sdiv_t,
      al_t, be_t)


def kernel(xyz, points, alpha, beta, W1, b1, g1, bb1, W2, b2, g2, bb2):
    f32 = jnp.float32
    xyz_t = jnp.transpose(xyz, (0, 2, 1))
    xyz_v = xyz_t.reshape(B_, 3, 64, 128)

    nx, ny, nz, fps_g = _run_fps(xyz_v)
    new_xyz = jnp.stack([nx, ny, nz], axis=-1)

    idx = _run_knn(new_xyz, xyz_t)

    idx_all = jnp.concatenate(
        [idx.reshape(NROWS), fps_g.reshape(B_ * S_)], axis=0)
    table = jnp.pad(points.reshape(B_ * N_, C_), ((0, 0), (0, C_)))
    rows = _run_sc_gather(table, idx_all)[:, :C_]
    grouped4 = rows[:NROWS].reshape(B_, S_, K_, C_)
    npnts = rows[NROWS:].reshape(B_, S_, C_)

    ssq = _run_p2(grouped4)
    std = jnp.sqrt(ssq[:, 0, 0] / (NSTD - 1))
    sdiv_t = std + 1e-5

    al_t = alpha.reshape(1, C_)
    be_t = beta.reshape(1, C_)
    w1t = W1.T
    h1, s1, s2 = _run_p3(grouped4, npnts, sdiv_t, al_t, be_t,
                         w1t[:C_, :], w1t[C_:, :], b1.reshape(1, D2_))

    m1 = s1 / CNT
    v1 = s2 / CNT - m1 * m1
    sc1 = g1.reshape(1, D2_) / jnp.sqrt(v1 + 1e-5)
    a1_t = sc1
    c1_t = bb1.reshape(1, D2_) - sc1 * m1

    w2t = W2.T
    h2a, h2b, t1a, t1b, t2a, t2b = _run_p4(
        h1, a1_t, c1_t, w2t[:, :C_], w2t[:, C_:],
        b2.reshape(1, D2_)[:, :C_], b2.reshape(1, D2_)[:, C_:])
    t1 = jnp.concatenate([t1a, t1b], axis=1)
    t2 = jnp.concatenate([t2a, t2b], axis=1)

    m2 = t1 / CNT
    v2 = t2 / CNT - m2 * m2
    sc2 = g2.reshape(1, D2_) / jnp.sqrt(v2 + 1e-5)
    c2 = bb2.reshape(1, D2_) - sc2 * m2
    o1, o2 = _run_p5(h2a, h2b, sc2[:, :C_], c2[:, :C_], sc2[:, C_:],
                     c2[:, C_:], grouped4, npnts, sdiv_t, al_t, be_t)

    h = jnp.concatenate([o1, o2], axis=-1)
    return (new_xyz.astype(f32), h.astype(f32))


# speedup vs baseline: 1.1022x; 1.1022x over previous
import functools

import jax
import jax.numpy as jnp
from jax import lax
from jax.experimental import pallas as pl
from jax.experimental.pallas import tpu as pltpu
from jax.experimental.pallas import tpu_sc as plsc

B_, N_, C_ = 4, 8192, 64
S_, K_ = 1024, 24
D2_ = 2 * C_
NSC, NSUB = 2, 16
NW = NSC * NSUB
NROWS = B_ * S_ * K_
NALL = NROWS + B_ * S_
CNT = float(B_ * S_ * K_)
NSTD = S_ * K_ * C_


def _fps_body(xyz_ref, nx_ref, ny_ref, nz_ref, gidx_ref, dist_ref):
    X = xyz_ref[:, 0]
    Y = xyz_ref[:, 1]
    Z = xyz_ref[:, 2]
    io_s = lax.broadcasted_iota(jnp.int32, (1, 64, 128), 1)
    io_l = lax.broadcasted_iota(jnp.int32, (1, 64, 128), 2)
    pos = io_s * 128 + io_l
    io_p = lax.broadcasted_iota(jnp.int32, (4, 1024), 1)
    boff = lax.broadcasted_iota(jnp.int32, (4, 1), 0) * N_

    dist_ref[...] = jnp.full((4, 64, 128), 1e10, dtype=jnp.float32)

    def step(s, far):
        fs = far // 128
        fl = far % 128
        oh = (io_s == fs) & (io_l == fl)
        cx = jnp.sum(jnp.where(oh, X, 0.0), axis=(1, 2), keepdims=True)
        cy = jnp.sum(jnp.where(oh, Y, 0.0), axis=(1, 2), keepdims=True)
        cz = jnp.sum(jnp.where(oh, Z, 0.0), axis=(1, 2), keepdims=True)
        sel = io_p == s
        nx_ref[...] = jnp.where(sel, cx[:, :, 0], nx_ref[...])
        ny_ref[...] = jnp.where(sel, cy[:, :, 0], ny_ref[...])
        nz_ref[...] = jnp.where(sel, cz[:, :, 0], nz_ref[...])
        gidx_ref[...] = jnp.where(sel, far[:, :, 0] + boff, gidx_ref[...])
        dx = X - cx
        dy = Y - cy
        dz = Z - cz
        d = (dx * dx + dy * dy) + dz * dz
        dist = jnp.minimum(dist_ref[...], d)
        dist_ref[...] = dist
        m = jnp.max(dist, axis=(1, 2), keepdims=True)
        cand = jnp.where(dist == m, pos, jnp.int32(2 ** 30))
        return jnp.min(cand, axis=(1, 2), keepdims=True)

    lax.fori_loop(0, S_, step, jnp.zeros((4, 1, 1), jnp.int32))


def _run_fps(xyz_v):
    f32 = jnp.float32
    return pl.pallas_call(
        _fps_body,
        out_shape=(
            jax.ShapeDtypeStruct((4, 1024), f32),
            jax.ShapeDtypeStruct((4, 1024), f32),
            jax.ShapeDtypeStruct((4, 1024), f32),
            jax.ShapeDtypeStruct((4, 1024), jnp.int32),
        ),
        scratch_shapes=[pltpu.VMEM((4, 64, 128), f32)],
    )(xyz_v)


SQ = 128


def _knn_body(q_ref, xyz_ref, idx_ref):
    b = pl.program_id(0)
    q = q_ref[0]
    qx = q[:, 0:1]
    qy = q[:, 1:2]
    qz = q[:, 2:3]
    px = xyz_ref[0, 0:1, :]
    py = xyz_ref[0, 1:2, :]
    pz = xyz_ref[0, 2:3, :]
    qn = (qx * qx + qy * qy) + qz * qz
    pn = (px * px + py * py) + pz * pz
    bf = jnp.bfloat16
    f32 = jnp.float32
    qxb = qx.astype(bf).astype(f32)
    qyb = qy.astype(bf).astype(f32)
    qzb = qz.astype(bf).astype(f32)
    pxb = px.astype(bf).astype(f32)
    pyb = py.astype(bf).astype(f32)
    pzb = pz.astype(bf).astype(f32)
    qp = (qxb * pxb + qyb * pyb) + qzb * pzb
    d = (qn + pn) - 2.0 * qp
    li = lax.broadcasted_iota(jnp.int32, (SQ, N_), 1)
    boff = b * N_
    mprev = jnp.full((SQ, 1), -3e38, jnp.float32)
    aprev = jnp.full((SQ, 1), -1, jnp.int32)
    for j in range(K_):
        ge = (d > mprev) | ((d == mprev) & (li > aprev))
        m = jnp.min(jnp.where(ge, d, jnp.float32(3e38)), axis=1,
                    keepdims=True)
        cand = jnp.where(ge & (d == m), li, jnp.int32(2 ** 30))
        a = jnp.min(cand, axis=1, keepdims=True)
        idx_ref[0, :, j:j + 1] = a + boff
        mprev, aprev = m, a


def _run_knn(new_xyz, xyz_t):
    nst = S_ // SQ
    return pl.pallas_call(
        _knn_body,
        grid=(B_, nst),
        in_specs=[
            pl.BlockSpec((1, SQ, 3), lambda b, st: (b, st, 0)),
            pl.BlockSpec((1, 3, N_), lambda b, st: (b, 0, 0)),
        ],
        out_specs=pl.BlockSpec((1, SQ, K_), lambda b, st: (b, st, 0)),
        out_shape=jax.ShapeDtypeStruct((B_, S_, K_), jnp.int32),
    )(new_xyz, xyz_t)


RPW = NALL // NW
CH = 800


def _sc_gather_body(table_hbm, idx_hbm, out_hbm, idx_v, rows_v, sem):
    wid = lax.axis_index("s") * NSC + lax.axis_index("c")
    base = wid * RPW
    for c in range(RPW // CH):
        off = base + c * CH
        pltpu.sync_copy(idx_hbm.at[pl.ds(off, CH)], idx_v)
        pltpu.async_copy(table_hbm.at[idx_v], rows_v, sem).wait()
        pltpu.sync_copy(rows_v, out_hbm.at[pl.ds(off, CH)])


def _run_sc_gather(table, idx_all):
    mesh = plsc.VectorSubcoreMesh(
        core_axis_name="c", subcore_axis_name="s",
        num_cores=NSC, num_subcores=NSUB)
    k = functools.partial(
        pl.kernel,
        out_type=jax.ShapeDtypeStruct((NALL, D2_), jnp.float32),
        mesh=mesh,
        scratch_types=[
            pltpu.VMEM((CH,), jnp.int32),
            pltpu.VMEM((CH, D2_), jnp.float32),
            pltpu.SemaphoreType.DMA,
        ],
    )(_sc_gather_body)
    return k(table, idx_all)


def _p2_body(g_ref, o_ref):
    st = pl.program_id(1)

    @pl.when(st == 0)
    def _():
        o_ref[...] = jnp.zeros_like(o_ref)

    g = g_ref[0]
    mean = jnp.mean(g, axis=1, keepdims=True)
    c = g - mean
    o_ref[...] = o_ref[...] + jnp.sum(c * c)


def _run_p2(grouped4):
    return pl.pallas_call(
        _p2_body,
        grid=(B_, 8),
        in_specs=[pl.BlockSpec((1, 128, K_, C_), lambda b, st: (b, st, 0, 0))],
        out_specs=pl.BlockSpec((1, 8, 128), lambda b, st: (b, 0, 0)),
        out_shape=jax.ShapeDtypeStruct((B_, 8, 128), jnp.float32),
    )(grouped4)


SCH = 128


def _p3_body(g_ref, np_ref, sdiv_ref, al_ref, be_ref, w1a_ref, w1b_ref,
             b1_ref, h1_ref, s1_ref, s2_ref):
    b = pl.program_id(0)
    sc = pl.program_id(1)

    @pl.when((b == 0) & (sc == 0))
    def _():
        s1_ref[...] = jnp.zeros_like(s1_ref)
        s2_ref[...] = jnp.zeros_like(s2_ref)

    g = g_ref[0]
    mean = jnp.mean(g, axis=1, keepdims=True)
    c = g - mean
    s = sdiv_ref[b]
    al = al_ref[...].reshape(1, 1, C_)
    be = be_ref[...].reshape(1, 1, C_)
    xg = al * (c / s) + be
    x1 = xg.reshape(SCH * K_, C_)
    x2 = jnp.broadcast_to(np_ref[0][:, None, :], (SCH, K_, C_)).reshape(
        SCH * K_, C_)
    h1 = (jnp.dot(x1, w1a_ref[...], preferred_element_type=jnp.float32)
          + jnp.dot(x2, w1b_ref[...], preferred_element_type=jnp.float32)
          ) + b1_ref[...]
    h1_ref[...] = h1
    s1_ref[...] = s1_ref[...] + jnp.sum(h1, axis=0, keepdims=True)
    s2_ref[...] = s2_ref[...] + jnp.sum(h1 * h1, axis=0, keepdims=True)


def _run_p3(grouped4, npnts, sdiv_t, al_t, be_t, w1a, w1b, b1_t):
    f32 = jnp.float32
    return pl.pallas_call(
        _p3_body,
        grid=(B_, S_ // SCH),
        in_specs=[
            pl.BlockSpec((1, SCH, K_, C_), lambda b, sc: (b, sc, 0, 0)),
            pl.BlockSpec((1, SCH, C_), lambda b, sc: (b, sc, 0)),
            pl.BlockSpec(memory_space=pltpu.SMEM),
            pl.BlockSpec((1, C_), lambda b, sc: (0, 0)),
            pl.BlockSpec((1, C_), lambda b, sc: (0, 0)),
            pl.BlockSpec((C_, D2_), lambda b, sc: (0, 0)),
            pl.BlockSpec((C_, D2_), lambda b, sc: (0, 0)),
            pl.BlockSpec((1, D2_), lambda b, sc: (0, 0)),
        ],
        out_specs=(
            pl.BlockSpec((SCH * K_, D2_),
                         lambda b, sc: (b * (S_ // SCH) + sc, 0)),
            pl.BlockSpec((1, D2_), lambda b, sc: (0, 0)),
            pl.BlockSpec((1, D2_), lambda b, sc: (0, 0)),
        ),
        out_shape=(
            jax.ShapeDtypeStruct((NROWS, D2_), f32),
            jax.ShapeDtypeStruct((1, D2_), f32),
            jax.ShapeDtypeStruct((1, D2_), f32),
        ),
    )(grouped4, npnts, sdiv_t, al_t, be_t, w1a, w1b, b1_t)


def _p4_body(h1_ref, a1_ref, c1_ref, w2a_ref, w2b_ref, b2a_ref, b2b_ref,
             h2a_ref, h2b_ref, t1a_ref, t1b_ref, t2a_ref, t2b_ref):
    i = pl.program_id(0)

    @pl.when(i == 0)
    def _():
        t1a_ref[...] = jnp.zeros_like(t1a_ref)
        t1b_ref[...] = jnp.zeros_like(t1b_ref)
        t2a_ref[...] = jnp.zeros_like(t2a_ref)
        t2b_ref[...] = jnp.zeros_like(t2b_ref)

    h1 = h1_ref[...]
    hn = a1_ref[...] * h1 + c1_ref[...]
    l = jnp.where(hn >= 0, hn, 0.01 * hn)
    h2a = jnp.dot(l, w2a_ref[...], preferred_element_type=jnp.float32) \
        + b2a_ref[...]
    h2b = jnp.dot(l, w2b_ref[...], preferred_element_type=jnp.float32) \
        + b2b_ref[...]
    h2a_ref[...] = h2a
    h2b_ref[...] = h2b
    t1a_ref[...] = t1a_ref[...] + jnp.sum(h2a, axis=0, keepdims=True)
    t1b_ref[...] = t1b_ref[...] + jnp.sum(h2b, axis=0, keepdims=True)
    t2a_ref[...] = t2a_ref[...] + jnp.sum(h2a * h2a, axis=0, keepdims=True)
    t2b_ref[...] = t2b_ref[...] + jnp.sum(h2b * h2b, axis=0, keepdims=True)


def _run_p4(h1, a1_t, c1_t, w2a, w2b, b2a_t, b2b_t):
    f32 = jnp.float32
    nt = NROWS // (SCH * K_)
    return pl.pallas_call(
        _p4_body,
        grid=(nt,),
        in_specs=[
            pl.BlockSpec((SCH * K_, D2_), lambda i: (i, 0)),
            pl.BlockSpec((1, D2_), lambda i: (0, 0)),
            pl.BlockSpec((1, D2_), lambda i: (0, 0)),
            pl.BlockSpec((D2_, C_), lambda i: (0, 0)),
            pl.BlockSpec((D2_, C_), lambda i: (0, 0)),
            pl.BlockSpec((1, C_), lambda i: (0, 0)),
            pl.BlockSpec((1, C_), lambda i: (0, 0)),
        ],
        out_specs=(
            pl.BlockSpec((SCH * K_, C_), lambda i: (i, 0)),
            pl.BlockSpec((SCH * K_, C_), lambda i: (i, 0)),
            pl.BlockSpec((1, C_), lambda i: (0, 0)),
            pl.BlockSpec((1, C_), lambda i: (0, 0)),
            pl.BlockSpec((1, C_), lambda i: (0, 0)),
            pl.BlockSpec((1, C_), lambda i: (0, 0)),
        ),
        out_shape=(
            jax.ShapeDtypeStruct((NROWS, C_), f32),
            jax.ShapeDtypeStruct((NROWS, C_), f32),
            jax.ShapeDtypeStruct((1, C_), f32),
            jax.ShapeDtypeStruct((1, C_), f32),
            jax.ShapeDtypeStruct((1, C_), f32),
            jax.ShapeDtypeStruct((1, C_), f32),
        ),
    )(h1, a1_t, c1_t, w2a, w2b, b2a_t, b2b_t)


def _p5_body(h2a_ref, h2b_ref, a2a_ref, c2a_ref, a2b_ref, c2b_ref,
             g_ref, np_ref, sdiv_ref, al_ref, be_ref, o1_ref, o2_ref):
    b = pl.program_id(0)
    g = g_ref[0]
    mean = jnp.mean(g, axis=1, keepdims=True)
    c = g - mean
    s = sdiv_ref[b]
    al = al_ref[...].reshape(1, 1, C_)
    be = be_ref[...].reshape(1, 1, C_)
    xg = al * (c / s) + be
    x1 = xg.reshape(SCH * K_, C_)
    x2 = jnp.broadcast_to(np_ref[0][:, None, :], (SCH, K_, C_)).reshape(
        SCH * K_, C_)
    ra = a2a_ref[...] * h2a_ref[...] + c2a_ref[...] + x1
    rb = a2b_ref[...] * h2b_ref[...] + c2b_ref[...] + x2
    la = jnp.where(ra >= 0, ra, 0.01 * ra).reshape(SCH, K_, C_)
    lb = jnp.where(rb >= 0, rb, 0.01 * rb).reshape(SCH, K_, C_)
    o1_ref[0] = jnp.max(la, axis=1)
    o2_ref[0] = jnp.max(lb, axis=1)


def _run_p5(h2a, h2b, a2a_t, c2a_t, a2b_t, c2b_t, grouped4, npnts, sdiv_t,
            al_t, be_t):
    f32 = jnp.float32
    nsc = S_ // SCH
    small = lambda b, sc: (0, 0)
    return pl.pallas_call(
        _p5_body,
        grid=(B_, nsc),
        in_specs=[
            pl.BlockSpec((SCH * K_, C_), lambda b, sc: (b * nsc + sc, 0)),
            pl.BlockSpec((SCH * K_, C_), lambda b, sc: (b * nsc + sc, 0)),
            pl.BlockSpec((1, C_), small),
            pl.BlockSpec((1, C_), small),
            pl.BlockSpec((1, C_), small),
            pl.BlockSpec((1, C_), small),
            pl.BlockSpec((1, SCH, K_, C_), lambda b, sc: (b, sc, 0, 0)),
            pl.BlockSpec((1, SCH, C_), lambda b, sc: (b, sc, 0)),
            pl.BlockSpec(memory_space=pltpu.SMEM),
            pl.BlockSpec((1, C_), small),
            pl.BlockSpec((1, C_), small),
        ],
        out_specs=(
            pl.BlockSpec((1, SCH, C_), lambda b, sc: (b, sc, 0)),
            pl.BlockSpec((1, SCH, C_), lambda b, sc: (b, sc, 0)),
        ),
        out_shape=(
            jax.ShapeDtypeStruct((B_, S_, C_), f32),
            jax.ShapeDtypeStruct((B_, S_, C_), f32),
        ),
    )(h2a, h2b, a2a_t, c2a_t, a2b_t, c2b_t, grouped4, npnts, sdiv_t,
      al_t, be_t)


def kernel(xyz, points, alpha, beta, W1, b1, g1, bb1, W2, b2, g2, bb2):
    f32 = jnp.float32
    xyz_t = jnp.transpose(xyz, (0, 2, 1))
    xyz_v = xyz_t.reshape(B_, 3, 64, 128)

    nx, ny, nz, fps_g = _run_fps(xyz_v)
    new_xyz = jnp.stack([nx, ny, nz], axis=-1)

    idx = _run_knn(new_xyz, xyz_t)

    idx_all = jnp.concatenate(
        [idx.reshape(NROWS), fps_g.reshape(B_ * S_)], axis=0)
    table = jnp.pad(points.reshape(B_ * N_, C_), ((0, 0), (0, C_)))
    rows = _run_sc_gather(table, idx_all)[:, :C_]
    grouped4 = rows[:NROWS].reshape(B_, S_, K_, C_)
    npnts = rows[NROWS:].reshape(B_, S_, C_)

    ssq = _run_p2(grouped4)
    std = jnp.sqrt(ssq[:, 0, 0] / (NSTD - 1))
    sdiv_t = std + 1e-5

    al_t = alpha.reshape(1, C_)
    be_t = beta.reshape(1, C_)
    w1t = W1.T
    h1, s1, s2 = _run_p3(grouped4, npnts, sdiv_t, al_t, be_t,
                         w1t[:C_, :], w1t[C_:, :], b1.reshape(1, D2_))

    m1 = s1 / CNT
    v1 = s2 / CNT - m1 * m1
    sc1 = g1.reshape(1, D2_) / jnp.sqrt(v1 + 1e-5)
    a1_t = sc1
    c1_t = bb1.reshape(1, D2_) - sc1 * m1

    w2t = W2.T
    h2a, h2b, t1a, t1b, t2a, t2b = _run_p4(
        h1, a1_t, c1_t, w2t[:, :C_], w2t[:, C_:],
        b2.reshape(1, D2_)[:, :C_], b2.reshape(1, D2_)[:, C_:])
    t1 = jnp.concatenate([t1a, t1b], axis=1)
    t2 = jnp.concatenate([t2a, t2b], axis=1)

    m2 = t1 / CNT
    v2 = t2 / CNT - m2 * m2
    sc2 = g2.reshape(1, D2_) / jnp.sqrt(v2 + 1e-5)
    c2 = bb2.reshape(1, D2_) - sc2 * m2
    o1, o2 = _run_p5(h2a, h2b, sc2[:, :C_], c2[:, :C_], sc2[:, C_:],
                     c2[:, C_:], grouped4, npnts, sdiv_t, al_t, be_t)

    h = jnp.concatenate([o1, o2], axis=-1)
    return (new_xyz.astype(f32), h.astype(f32))


# speedup vs baseline: 1.6870x; 1.5306x over previous
import functools

import jax
import jax.numpy as jnp
from jax import lax
from jax.experimental import pallas as pl
from jax.experimental.pallas import tpu as pltpu
from jax.experimental.pallas import tpu_sc as plsc

B_, N_, C_ = 4, 8192, 64
S_, K_ = 1024, 24
D2_ = 2 * C_
NSC, NSUB = 2, 16
NW = NSC * NSUB
NROWS = B_ * S_ * K_
NALL = NROWS + B_ * S_
CNT = float(B_ * S_ * K_)
NSTD = S_ * K_ * C_


def _fps_body(xyz_ref, nx_ref, ny_ref, nz_ref, gidx_ref, dist_ref):
    X = xyz_ref[:, 0]
    Y = xyz_ref[:, 1]
    Z = xyz_ref[:, 2]
    io_s = lax.broadcasted_iota(jnp.int32, (1, 64, 128), 1)
    io_l = lax.broadcasted_iota(jnp.int32, (1, 64, 128), 2)
    pos = io_s * 128 + io_l
    io_p = lax.broadcasted_iota(jnp.int32, (4, 1024), 1)
    boff = lax.broadcasted_iota(jnp.int32, (4, 1), 0) * N_

    dist_ref[...] = jnp.full((4, 64, 128), 1e10, dtype=jnp.float32)

    def step(s, far):
        fs = far // 128
        fl = far % 128
        oh = (io_s == fs) & (io_l == fl)
        cx = jnp.sum(jnp.where(oh, X, 0.0), axis=(1, 2), keepdims=True)
        cy = jnp.sum(jnp.where(oh, Y, 0.0), axis=(1, 2), keepdims=True)
        cz = jnp.sum(jnp.where(oh, Z, 0.0), axis=(1, 2), keepdims=True)
        sel = io_p == s
        nx_ref[...] = jnp.where(sel, cx[:, :, 0], nx_ref[...])
        ny_ref[...] = jnp.where(sel, cy[:, :, 0], ny_ref[...])
        nz_ref[...] = jnp.where(sel, cz[:, :, 0], nz_ref[...])
        gidx_ref[...] = jnp.where(sel, far[:, :, 0] + boff, gidx_ref[...])
        dx = X - cx
        dy = Y - cy
        dz = Z - cz
        d = (dx * dx + dy * dy) + dz * dz
        dist = jnp.minimum(dist_ref[...], d)
        dist_ref[...] = dist
        m = jnp.max(dist, axis=(1, 2), keepdims=True)
        cand = jnp.where(dist == m, pos, jnp.int32(2 ** 30))
        return jnp.min(cand, axis=(1, 2), keepdims=True)

    lax.fori_loop(0, S_, step, jnp.zeros((4, 1, 1), jnp.int32))


def _run_fps(xyz_v):
    f32 = jnp.float32
    return pl.pallas_call(
        _fps_body,
        out_shape=(
            jax.ShapeDtypeStruct((4, 1024), f32),
            jax.ShapeDtypeStruct((4, 1024), f32),
            jax.ShapeDtypeStruct((4, 1024), f32),
            jax.ShapeDtypeStruct((4, 1024), jnp.int32),
        ),
        scratch_shapes=[pltpu.VMEM((4, 64, 128), f32)],
    )(xyz_v)


SQ = 128


def _knn_body(q_ref, xyz_ref, idx_ref):
    b = pl.program_id(0)
    q = q_ref[0]
    qx = q[:, 0:1]
    qy = q[:, 1:2]
    qz = q[:, 2:3]
    px = xyz_ref[0, 0:1, :]
    py = xyz_ref[0, 1:2, :]
    pz = xyz_ref[0, 2:3, :]
    qn = (qx * qx + qy * qy) + qz * qz
    pn = (px * px + py * py) + pz * pz
    bf = jnp.bfloat16
    f32 = jnp.float32
    qxb = qx.astype(bf).astype(f32)
    qyb = qy.astype(bf).astype(f32)
    qzb = qz.astype(bf).astype(f32)
    pxb = px.astype(bf).astype(f32)
    pyb = py.astype(bf).astype(f32)
    pzb = pz.astype(bf).astype(f32)
    qp = (qxb * pxb + qyb * pyb) + qzb * pzb
    d = (qn + pn) - 2.0 * qp
    li = lax.broadcasted_iota(jnp.int32, (SQ, N_), 1)
    boff = b * N_
    for j in range(K_):
        m = jnp.min(d, axis=1, keepdims=True)
        cand = jnp.where(d == m, li, jnp.int32(2 ** 30))
        a = jnp.min(cand, axis=1, keepdims=True)
        idx_ref[0, :, j:j + 1] = a + boff
        d = jnp.where(li == a, jnp.float32(3e38), d)


def _run_knn(new_xyz, xyz_t):
    nst = S_ // SQ
    return pl.pallas_call(
        _knn_body,
        grid=(B_, nst),
        in_specs=[
            pl.BlockSpec((1, SQ, 3), lambda b, st: (b, st, 0)),
            pl.BlockSpec((1, 3, N_), lambda b, st: (b, 0, 0)),
        ],
        out_specs=pl.BlockSpec((1, SQ, K_), lambda b, st: (b, st, 0)),
        out_shape=jax.ShapeDtypeStruct((B_, S_, K_), jnp.int32),
    )(new_xyz, xyz_t)


RPW = NALL // NW
CH = 800


def _sc_gather_body(table_hbm, idx_hbm, out_hbm, idx_v, rows_v, sem):
    wid = lax.axis_index("s") * NSC + lax.axis_index("c")
    base = wid * RPW
    for c in range(RPW // CH):
        off = base + c * CH
        pltpu.sync_copy(idx_hbm.at[pl.ds(off, CH)], idx_v)
        pltpu.async_copy(table_hbm.at[idx_v], rows_v, sem).wait()
        pltpu.sync_copy(rows_v, out_hbm.at[pl.ds(off, CH)])


def _run_sc_gather(table, idx_all):
    mesh = plsc.VectorSubcoreMesh(
        core_axis_name="c", subcore_axis_name="s",
        num_cores=NSC, num_subcores=NSUB)
    k = functools.partial(
        pl.kernel,
        out_type=jax.ShapeDtypeStruct((NALL, D2_), jnp.float32),
        mesh=mesh,
        scratch_types=[
            pltpu.VMEM((CH,), jnp.int32),
            pltpu.VMEM((CH, D2_), jnp.float32),
            pltpu.SemaphoreType.DMA,
        ],
    )(_sc_gather_body)
    return k(table, idx_all)


def _p2_body(g_ref, o_ref):
    st = pl.program_id(1)

    @pl.when(st == 0)
    def _():
        o_ref[...] = jnp.zeros_like(o_ref)

    g = g_ref[0]
    mean = jnp.mean(g, axis=1, keepdims=True)
    c = g - mean
    o_ref[...] = o_ref[...] + jnp.sum(c * c)


def _run_p2(grouped4):
    return pl.pallas_call(
        _p2_body,
        grid=(B_, 8),
        in_specs=[pl.BlockSpec((1, 128, K_, C_), lambda b, st: (b, st, 0, 0))],
        out_specs=pl.BlockSpec((1, 8, 128), lambda b, st: (b, 0, 0)),
        out_shape=jax.ShapeDtypeStruct((B_, 8, 128), jnp.float32),
    )(grouped4)


SCH = 128


def _p3_body(g_ref, np_ref, sdiv_ref, al_ref, be_ref, w1a_ref, w1b_ref,
             b1_ref, h1_ref, s1_ref, s2_ref):
    b = pl.program_id(0)
    sc = pl.program_id(1)

    @pl.when((b == 0) & (sc == 0))
    def _():
        s1_ref[...] = jnp.zeros_like(s1_ref)
        s2_ref[...] = jnp.zeros_like(s2_ref)

    g = g_ref[0]
    mean = jnp.mean(g, axis=1, keepdims=True)
    c = g - mean
    s = sdiv_ref[b]
    al = al_ref[...].reshape(1, 1, C_)
    be = be_ref[...].reshape(1, 1, C_)
    xg = al * (c / s) + be
    x1 = xg.reshape(SCH * K_, C_)
    x2 = jnp.broadcast_to(np_ref[0][:, None, :], (SCH, K_, C_)).reshape(
        SCH * K_, C_)
    h1 = (jnp.dot(x1, w1a_ref[...], preferred_element_type=jnp.float32)
          + jnp.dot(x2, w1b_ref[...], preferred_element_type=jnp.float32)
          ) + b1_ref[...]
    h1_ref[...] = h1
    s1_ref[...] = s1_ref[...] + jnp.sum(h1, axis=0, keepdims=True)
    s2_ref[...] = s2_ref[...] + jnp.sum(h1 * h1, axis=0, keepdims=True)


def _run_p3(grouped4, npnts, sdiv_t, al_t, be_t, w1a, w1b, b1_t):
    f32 = jnp.float32
    return pl.pallas_call(
        _p3_body,
        grid=(B_, S_ // SCH),
        in_specs=[
            pl.BlockSpec((1, SCH, K_, C_), lambda b, sc: (b, sc, 0, 0)),
            pl.BlockSpec((1, SCH, C_), lambda b, sc: (b, sc, 0)),
            pl.BlockSpec(memory_space=pltpu.SMEM),
            pl.BlockSpec((1, C_), lambda b, sc: (0, 0)),
            pl.BlockSpec((1, C_), lambda b, sc: (0, 0)),
            pl.BlockSpec((C_, D2_), lambda b, sc: (0, 0)),
            pl.BlockSpec((C_, D2_), lambda b, sc: (0, 0)),
            pl.BlockSpec((1, D2_), lambda b, sc: (0, 0)),
        ],
        out_specs=(
            pl.BlockSpec((SCH * K_, D2_),
                         lambda b, sc: (b * (S_ // SCH) + sc, 0)),
            pl.BlockSpec((1, D2_), lambda b, sc: (0, 0)),
            pl.BlockSpec((1, D2_), lambda b, sc: (0, 0)),
        ),
        out_shape=(
            jax.ShapeDtypeStruct((NROWS, D2_), f32),
            jax.ShapeDtypeStruct((1, D2_), f32),
            jax.ShapeDtypeStruct((1, D2_), f32),
        ),
    )(grouped4, npnts, sdiv_t, al_t, be_t, w1a, w1b, b1_t)


def _p4_body(h1_ref, a1_ref, c1_ref, w2a_ref, w2b_ref, b2a_ref, b2b_ref,
             h2a_ref, h2b_ref, t1a_ref, t1b_ref, t2a_ref, t2b_ref):
    i = pl.program_id(0)

    @pl.when(i == 0)
    def _():
        t1a_ref[...] = jnp.zeros_like(t1a_ref)
        t1b_ref[...] = jnp.zeros_like(t1b_ref)
        t2a_ref[...] = jnp.zeros_like(t2a_ref)
        t2b_ref[...] = jnp.zeros_like(t2b_ref)

    h1 = h1_ref[...]
    hn = a1_ref[...] * h1 + c1_ref[...]
    l = jnp.where(hn >= 0, hn, 0.01 * hn)
    h2a = jnp.dot(l, w2a_ref[...], preferred_element_type=jnp.float32) \
        + b2a_ref[...]
    h2b = jnp.dot(l, w2b_ref[...], preferred_element_type=jnp.float32) \
        + b2b_ref[...]
    h2a_ref[...] = h2a
    h2b_ref[...] = h2b
    t1a_ref[...] = t1a_ref[...] + jnp.sum(h2a, axis=0, keepdims=True)
    t1b_ref[...] = t1b_ref[...] + jnp.sum(h2b, axis=0, keepdims=True)
    t2a_ref[...] = t2a_ref[...] + jnp.sum(h2a * h2a, axis=0, keepdims=True)
    t2b_ref[...] = t2b_ref[...] + jnp.sum(h2b * h2b, axis=0, keepdims=True)


def _run_p4(h1, a1_t, c1_t, w2a, w2b, b2a_t, b2b_t):
    f32 = jnp.float32
    nt = NROWS // (SCH * K_)
    return pl.pallas_call(
        _p4_body,
        grid=(nt,),
        in_specs=[
            pl.BlockSpec((SCH * K_, D2_), lambda i: (i, 0)),
            pl.BlockSpec((1, D2_), lambda i: (0, 0)),
            pl.BlockSpec((1, D2_), lambda i: (0, 0)),
            pl.BlockSpec((D2_, C_), lambda i: (0, 0)),
            pl.BlockSpec((D2_, C_), lambda i: (0, 0)),
            pl.BlockSpec((1, C_), lambda i: (0, 0)),
            pl.BlockSpec((1, C_), lambda i: (0, 0)),
        ],
        out_specs=(
            pl.BlockSpec((SCH * K_, C_), lambda i: (i, 0)),
            pl.BlockSpec((SCH * K_, C_), lambda i: (i, 0)),
            pl.BlockSpec((1, C_), lambda i: (0, 0)),
            pl.BlockSpec((1, C_), lambda i: (0, 0)),
            pl.BlockSpec((1, C_), lambda i: (0, 0)),
            pl.BlockSpec((1, C_), lambda i: (0, 0)),
        ),
        out_shape=(
            jax.ShapeDtypeStruct((NROWS, C_), f32),
            jax.ShapeDtypeStruct((NROWS, C_), f32),
            jax.ShapeDtypeStruct((1, C_), f32),
            jax.ShapeDtypeStruct((1, C_), f32),
            jax.ShapeDtypeStruct((1, C_), f32),
            jax.ShapeDtypeStruct((1, C_), f32),
        ),
    )(h1, a1_t, c1_t, w2a, w2b, b2a_t, b2b_t)


def _p5_body(h2a_ref, h2b_ref, a2a_ref, c2a_ref, a2b_ref, c2b_ref,
             g_ref, np_ref, sdiv_ref, al_ref, be_ref, o1_ref, o2_ref):
    b = pl.program_id(0)
    g = g_ref[0]
    mean = jnp.mean(g, axis=1, keepdims=True)
    c = g - mean
    s = sdiv_ref[b]
    al = al_ref[...].reshape(1, 1, C_)
    be = be_ref[...].reshape(1, 1, C_)
    xg = al * (c / s) + be
    x1 = xg.reshape(SCH * K_, C_)
    x2 = jnp.broadcast_to(np_ref[0][:, None, :], (SCH, K_, C_)).reshape(
        SCH * K_, C_)
    ra = a2a_ref[...] * h2a_ref[...] + c2a_ref[...] + x1
    rb = a2b_ref[...] * h2b_ref[...] + c2b_ref[...] + x2
    la = jnp.where(ra >= 0, ra, 0.01 * ra).reshape(SCH, K_, C_)
    lb = jnp.where(rb >= 0, rb, 0.01 * rb).reshape(SCH, K_, C_)
    o1_ref[0] = jnp.max(la, axis=1)
    o2_ref[0] = jnp.max(lb, axis=1)


def _run_p5(h2a, h2b, a2a_t, c2a_t, a2b_t, c2b_t, grouped4, npnts, sdiv_t,
            al_t, be_t):
    f32 = jnp.float32
    nsc = S_ // SCH
    small = lambda b, sc: (0, 0)
    return pl.pallas_call(
        _p5_body,
        grid=(B_, nsc),
        in_specs=[
            pl.BlockSpec((SCH * K_, C_), lambda b, sc: (b * nsc + sc, 0)),
            pl.BlockSpec((SCH * K_, C_), lambda b, sc: (b * nsc + sc, 0)),
            pl.BlockSpec((1, C_), small),
            pl.BlockSpec((1, C_), small),
            pl.BlockSpec((1, C_), small),
            pl.BlockSpec((1, C_), small),
            pl.BlockSpec((1, SCH, K_, C_), lambda b, sc: (b, sc, 0, 0)),
            pl.BlockSpec((1, SCH, C_), lambda b, sc: (b, sc, 0)),
            pl.BlockSpec(memory_space=pltpu.SMEM),
            pl.BlockSpec((1, C_), small),
            pl.BlockSpec((1, C_), small),
        ],
        out_specs=(
            pl.BlockSpec((1, SCH, C_), lambda b, sc: (b, sc, 0)),
            pl.BlockSpec((1, SCH, C_), lambda b, sc: (b, sc, 0)),
        ),
        out_shape=(
            jax.ShapeDtypeStruct((B_, S_, C_), f32),
            jax.ShapeDtypeStruct((B_, S_, C_), f32),
        ),
    )(h2a, h2b, a2a_t, c2a_t, a2b_t, c2b_t, grouped4, npnts, sdiv_t,
      al_t, be_t)


def kernel(xyz, points, alpha, beta, W1, b1, g1, bb1, W2, b2, g2, bb2):
    f32 = jnp.float32
    xyz_t = jnp.transpose(xyz, (0, 2, 1))
    xyz_v = xyz_t.reshape(B_, 3, 64, 128)

    nx, ny, nz, fps_g = _run_fps(xyz_v)
    new_xyz = jnp.stack([nx, ny, nz], axis=-1)

    idx = _run_knn(new_xyz, xyz_t)

    idx_all = jnp.concatenate(
        [idx.reshape(NROWS), fps_g.reshape(B_ * S_)], axis=0)
    table = jnp.pad(points.reshape(B_ * N_, C_), ((0, 0), (0, C_)))
    rows = _run_sc_gather(table, idx_all)[:, :C_]
    grouped4 = rows[:NROWS].reshape(B_, S_, K_, C_)
    npnts = rows[NROWS:].reshape(B_, S_, C_)

    ssq = _run_p2(grouped4)
    std = jnp.sqrt(ssq[:, 0, 0] / (NSTD - 1))
    sdiv_t = std + 1e-5

    al_t = alpha.reshape(1, C_)
    be_t = beta.reshape(1, C_)
    w1t = W1.T
    h1, s1, s2 = _run_p3(grouped4, npnts, sdiv_t, al_t, be_t,
                         w1t[:C_, :], w1t[C_:, :], b1.reshape(1, D2_))

    m1 = s1 / CNT
    v1 = s2 / CNT - m1 * m1
    sc1 = g1.reshape(1, D2_) / jnp.sqrt(v1 + 1e-5)
    a1_t = sc1
    c1_t = bb1.reshape(1, D2_) - sc1 * m1

    w2t = W2.T
    h2a, h2b, t1a, t1b, t2a, t2b = _run_p4(
        h1, a1_t, c1_t, w2t[:, :C_], w2t[:, C_:],
        b2.reshape(1, D2_)[:, :C_], b2.reshape(1, D2_)[:, C_:])
    t1 = jnp.concatenate([t1a, t1b], axis=1)
    t2 = jnp.concatenate([t2a, t2b], axis=1)

    m2 = t1 / CNT
    v2 = t2 / CNT - m2 * m2
    sc2 = g2.reshape(1, D2_) / jnp.sqrt(v2 + 1e-5)
    c2 = bb2.reshape(1, D2_) - sc2 * m2
    o1, o2 = _run_p5(h2a, h2b, sc2[:, :C_], c2[:, :C_], sc2[:, C_:],
                     c2[:, C_:], grouped4, npnts, sdiv_t, al_t, be_t)

    h = jnp.concatenate([o1, o2], axis=-1)
    return (new_xyz.astype(f32), h.astype(f32))
